# baseline (speedup 1.0000x reference)
import functools

import jax
import jax.numpy as jnp
from jax import lax
from jax.experimental import pallas as pl
from jax.experimental.pallas import tpu as pltpu
from jax.experimental.pallas import tpu_sc as plsc

EPS = 1e-5
_N = 10000
_NP = 10240
_ZC = 640
_OC = 632
_PO = 16 * _OC
_RB = 1000
_NRB = _N // _RB
_NC, _NS = 2, 16
_NW = _NC * _NS

_EA_DEG_B = 44
_EAP = _NW * _EA_DEG_B * 128
_SEG_B = 88
_EDP2 = 16 * 160 * 128


def _mesh():
    return plsc.VectorSubcoreMesh(core_axis_name="c", subcore_axis_name="s")


def _tc1_body(deg2_ref, x_ref, w1_ref, m1_ref, dis_ref):
    degb = deg2_ref[0] + deg2_ref[1]
    dis = lax.rsqrt(degb[:, 0:1])
    dis_ref[...] = dis
    xs = x_ref[...] * dis
    m1_ref[0] = jnp.dot(xs, w1_ref[...], preferred_element_type=jnp.float32)


def _tc1(deg2, X, W1):
    return pl.pallas_call(
        _tc1_body,
        grid=(_NRB, 4),
        in_specs=[
            pl.BlockSpec((2, _RB, 16), lambda i, j: (0, i, 0)),
            pl.BlockSpec((_RB, 256), lambda i, j: (i, 0)),
            pl.BlockSpec((256, 128), lambda i, j: (0, j)),
        ],
        out_specs=[
            pl.BlockSpec((1, _RB, 128), lambda i, j: (j, i, 0)),
            pl.BlockSpec((_RB, 1), lambda i, j: (i, 0)),
        ],
        out_shape=[
            jax.ShapeDtypeStruct((4, _N, 128), jnp.float32),
            jax.ShapeDtypeStruct((_N, 1), jnp.float32),
        ],
    )(deg2, X, W1)


def _tc2a_body(hr_ref, dis_ref, t_ref, st_ref, acc):
    i = pl.program_id(1)
    t = (hr_ref[0] * dis_ref[...]) ** 2
    t_ref[0] = t

    @pl.when(i == 0)
    def _():
        acc[...] = jnp.zeros_like(acc)

    acc[0:1] += jnp.sum(t, axis=0, keepdims=True)
    acc[1:2] += jnp.sum(t * t, axis=0, keepdims=True)

    @pl.when(i == _NRB - 1)
    def _():
        st_ref[0] = acc[...]


def _tc2a(hr, dis):
    return pl.pallas_call(
        _tc2a_body,
        grid=(4, _NRB),
        in_specs=[
            pl.BlockSpec((1, _RB, 128), lambda k, i: (k, i, 0)),
            pl.BlockSpec((_RB, 1), lambda k, i: (i, 0)),
        ],
        out_specs=[
            pl.BlockSpec((1, _RB, 128), lambda k, i: (k, i, 0)),
            pl.BlockSpec((1, 8, 128), lambda k, i: (k, 0, 0)),
        ],
        out_shape=[
            jax.ShapeDtypeStruct((4, _N, 128), jnp.float32),
            jax.ShapeDtypeStruct((4, 8, 128), jnp.float32),
        ],
        scratch_shapes=[pltpu.VMEM((8, 128), jnp.float32)],
    )(hr, dis)


def _tc2b_body(t_ref, st_ref, g_ref, b_ref, dis_ref, w2_ref, m2_ref):
    k = pl.program_id(2)
    sm = st_ref[0, 0:1, :] * (1.0 / _N)
    sq = st_ref[0, 1:2, :] * (1.0 / _N)
    inv = lax.rsqrt(sq - sm * sm + EPS)
    hb = ((t_ref[0] - sm) * inv * g_ref[0, 0:1, :] + b_ref[0, 0:1, :]) * dis_ref[...]
    part = jnp.dot(hb, w2_ref[0], preferred_element_type=jnp.float32)

    @pl.when(k == 0)
    def _():
        m2_ref[0] = part

    @pl.when(k > 0)
    def _():
        m2_ref[0] += part


def _tc2b(t, st, gamma4, beta4, dis, w2r):
    return pl.pallas_call(
        _tc2b_body,
        grid=(_NRB, 2, 4),
        in_specs=[
            pl.BlockSpec((1, _RB, 128), lambda i, jo, k: (k, i, 0)),
            pl.BlockSpec((1, 8, 128), lambda i, jo, k: (k, 0, 0)),
            pl.BlockSpec((1, 8, 128), lambda i, jo, k: (k, 0, 0)),
            pl.BlockSpec((1, 8, 128), lambda i, jo, k: (k, 0, 0)),
            pl.BlockSpec((_RB, 1), lambda i, jo, k: (i, 0)),
            pl.BlockSpec((1, 128, 128), lambda i, jo, k: (k, 0, jo)),
        ],
        out_specs=pl.BlockSpec((1, _RB, 128), lambda i, jo, k: (jo, i, 0)),
        out_shape=jax.ShapeDtypeStruct((2, _N, 128), jnp.float32),
    )(t, st, gamma4, beta4, dis, w2r)


def _sc_deg(rows3, zeros16, ones16):
    @functools.partial(
        pl.kernel,
        out_type=jax.ShapeDtypeStruct((_NC * _PO, 16), jnp.float32),
        mesh=_mesh(),
        scratch_types=[
            pltpu.VMEM((_EA_DEG_B, 128), jnp.int32),
            pltpu.VMEM((128, 16), jnp.float32),
            pltpu.VMEM_SHARED((_NP, 16), jnp.float32),
        ],
    )
    def run(rows_h, z16_h, o16_h, deg_h, idx_v, ones_v, acc):
        c = lax.axis_index("c")
        s = lax.axis_index("s")
        wid = s * _NC + c
        pltpu.sync_copy(z16_h, acc.at[pl.ds(s * _ZC, _ZC)])
        pltpu.sync_copy(rows_h.at[wid], idx_v)
        pltpu.sync_copy(o16_h, ones_v)
        plsc.subcore_barrier()

        def bstep(b, carry):
            pltpu.sync_copy(ones_v, acc.at[idx_v.at[b]], add=True)
            return carry

        lax.fori_loop(0, _EA_DEG_B, bstep, 0)
        plsc.subcore_barrier()
        pltpu.sync_copy(acc.at[pl.ds(s * _OC, _OC)],
                        deg_h.at[pl.ds(c * _PO + s * _OC, _OC)])

    return run(rows3, zeros16, ones16).reshape(_NC, _PO, 16)


_SCH = 8


def _sc_segsum(nch, cols3, rows3, zerosb, mts):
    cpc = nch // _NC

    @functools.partial(
        pl.kernel,
        out_type=jax.ShapeDtypeStruct((nch * _PO, 128), jnp.float32),
        mesh=_mesh(),
        scratch_types=[
            pltpu.VMEM((_SCH, 128), jnp.int32),
            pltpu.VMEM((_SCH, 128), jnp.int32),
            pltpu.VMEM((128, 128), jnp.float32),
            pltpu.VMEM((128, 128), jnp.float32),
            pltpu.VMEM_SHARED((_NP, 128), jnp.float32),
            pltpu.SemaphoreType.DMA,
            pltpu.SemaphoreType.DMA,
            pltpu.SemaphoreType.DMA,
            pltpu.SemaphoreType.DMA,
        ],
    )
    def run(cols_h, rows_h, z_h, *rest):
        mt_hs = rest[:nch]
        out_h = rest[nch]
        cidx, ridx, g0, g1, acc, smg0, smg1, sms0, sms1 = rest[nch + 1:]
        c = lax.axis_index("c")
        s = lax.axis_index("s")
        for chunk in range(nch):
            my = (chunk // cpc) == c

            @pl.when(my)
            def _zero():
                pltpu.sync_copy(z_h, acc.at[pl.ds(s * _ZC, _ZC)])

            plsc.subcore_barrier()

            @pl.when(my)
            def _work():
                def q_step(q, cq):
                    cia = pltpu.async_copy(
                        cols_h.at[s, pl.ds(q * _SCH, _SCH)], cidx, smg0)
                    ria = pltpu.async_copy(
                        rows_h.at[s, pl.ds(q * _SCH, _SCH)], ridx, smg1)
                    cia.wait()
                    ria.wait()

                    def pair(i, cp):
                        b0 = 2 * i
                        b1 = 2 * i + 1
                        g0c = pltpu.async_copy(
                            mt_hs[chunk].at[cidx.at[b0]], g0, smg0)
                        g0c.wait()
                        g1c = pltpu.async_copy(
                            mt_hs[chunk].at[cidx.at[b1]], g1, smg1)
                        s0c = pltpu.async_copy(
                            g0, acc.at[ridx.at[b0]], sms0, add=True)
                        g1c.wait()
                        s0c.wait()
                        s1c = pltpu.async_copy(
                            g1, acc.at[ridx.at[b1]], sms1, add=True)
                        s1c.wait()
                        return cp

                    lax.fori_loop(0, _SCH // 2, pair, 0)
                    return cq

                lax.fori_loop(0, _SEG_B // _SCH, q_step, 0)

            plsc.subcore_barrier()

            @pl.when(my)
            def _out():
                pltpu.sync_copy(acc.at[pl.ds(s * _OC, _OC)],
                                out_h.at[pl.ds(chunk * _PO + s * _OC, _OC)])

    return run(cols3, rows3, zerosb, *mts).reshape(nch, _PO, 128)


def _tc3_body(zr_ref, dis_ref, z_ref):
    z_ref[0] = zr_ref[0] * dis_ref[...]


def _tc3(zr, dis_pad):
    return pl.pallas_call(
        _tc3_body,
        grid=(2, 16),
        in_specs=[
            pl.BlockSpec((1, _OC, 128), lambda k, i: (k, i, 0)),
            pl.BlockSpec((_OC, 1), lambda k, i: (i, 0)),
        ],
        out_specs=pl.BlockSpec((1, _OC, 128), lambda k, i: (k, i, 0)),
        out_shape=jax.ShapeDtypeStruct((2, _PO, 128), jnp.float32),
    )(zr, dis_pad)


_DB2 = 160
_DCH = 8
_EPS2 = _DB2 * 128
_ZT = _N - 15 * _OC


def _sc_decode(z0, z1, a3, b3):
    @functools.partial(
        pl.kernel,
        out_type=jax.ShapeDtypeStruct((_NC * _EDP2, 16), jnp.float32),
        mesh=_mesh(),
        scratch_types=[
            pltpu.VMEM((_DCH, 128), jnp.int32),
            pltpu.VMEM((_DCH, 128), jnp.int32),
            pltpu.VMEM((128, 128), jnp.float32),
            pltpu.VMEM((128, 128), jnp.float32),
            pltpu.VMEM((64, 16), jnp.float32),
            pltpu.VMEM_SHARED((_N, 128), jnp.float32),
            pltpu.SemaphoreType.DMA,
            pltpu.SemaphoreType.DMA,
        ],
    )
    def run(z0_h, z1_h, a_h, b_h, out_h, av, bv, ga, gb, prow, zsp,
            sema, semb):
        c = lax.axis_index("c")
        s = lax.axis_index("s")

        @pl.when(jnp.logical_and(c == 0, s < 15))
        def _():
            pltpu.sync_copy(z0_h.at[pl.ds(s * _OC, _OC)],
                            zsp.at[pl.ds(s * _OC, _OC)])

        @pl.when(jnp.logical_and(c == 0, s == 15))
        def _():
            pltpu.sync_copy(z0_h.at[pl.ds(15 * _OC, _ZT)],
                            zsp.at[pl.ds(15 * _OC, _ZT)])

        @pl.when(jnp.logical_and(c == 1, s < 15))
        def _():
            pltpu.sync_copy(z1_h.at[pl.ds(s * _OC, _OC)],
                            zsp.at[pl.ds(s * _OC, _OC)])

        @pl.when(jnp.logical_and(c == 1, s == 15))
        def _():
            pltpu.sync_copy(z1_h.at[pl.ds(15 * _OC, _ZT)],
                            zsp.at[pl.ds(15 * _OC, _ZT)])

        plsc.subcore_barrier()

        def chunk_step(ch, carry):
            ca = pltpu.async_copy(a_h.at[s, pl.ds(ch * _DCH, _DCH)], av, sema)
            cb = pltpu.async_copy(b_h.at[s, pl.ds(ch * _DCH, _DCH)], bv, semb)
            ca.wait()
            cb.wait()

            def batch(b, carry2):
                ga_c = pltpu.async_copy(zsp.at[av.at[b]], ga, sema)
                gb_c = pltpu.async_copy(zsp.at[bv.at[b]], gb, semb)
                ga_c.wait()
                gb_c.wait()

                for half in range(2):
                    def estep(e, cc):
                        eh = 64 * half + e
                        acc = ga[eh, pl.ds(0, 16)] * gb[eh, pl.ds(0, 16)]
                        for k in range(1, 8):
                            acc = acc + ga[eh, pl.ds(16 * k, 16)] * gb[eh, pl.ds(16 * k, 16)]
                        prow[e] = acc
                        return cc

                    lax.fori_loop(0, 64, estep, 0)
                    pltpu.sync_copy(
                        prow,
                        out_h.at[pl.ds(
                            c * _EDP2 + s * _EPS2
                            + 128 * (ch * _DCH + b) + 64 * half, 64)])
                return carry2

            lax.fori_loop(0, _DCH, batch, 0)
            return carry

        lax.fori_loop(0, _DB2 // _DCH, chunk_step, 0)

    return run(z0, z1, a3, b3)


_T4B = 2048


def _tc4_body(p0_ref, p1_ref, o_ref):
    tot = jnp.sum(p0_ref[...] + p1_ref[...], axis=1, keepdims=True)
    o_ref[...] = jax.nn.sigmoid(tot)


def _tc4(p0, p1):
    t = p0.shape[0]
    return pl.pallas_call(
        _tc4_body,
        grid=(t // _T4B,),
        in_specs=[
            pl.BlockSpec((_T4B, 16), lambda i: (i, 0)),
            pl.BlockSpec((_T4B, 16), lambda i: (i, 0)),
        ],
        out_specs=pl.BlockSpec((_T4B, 1), lambda i: (i, 0)),
        out_shape=jax.ShapeDtypeStruct((t, 1), jnp.float32),
    )(p0, p1)


def kernel(X, W1, W2, gamma, beta, adj_edge_index, pos_edge_index, neg_edge_index):
    E = pos_edge_index.shape[1]
    rows = adj_edge_index[0]
    cols = adj_edge_index[1]
    pad_a = _EAP - rows.shape[0]
    rows_p = jnp.concatenate([rows, jnp.full((pad_a,), _N, jnp.int32)])
    cols_p = jnp.concatenate([cols, jnp.zeros((pad_a,), jnp.int32)])
    rows32 = rows_p.reshape(_NW, _EA_DEG_B, 128)
    rows16 = rows_p.reshape(_NS, _SEG_B, 128)
    cols16 = cols_p.reshape(_NS, _SEG_B, 128)
    z16 = jnp.zeros((_ZC, 16), jnp.float32)
    ones16 = jnp.ones((128, 16), jnp.float32)
    z128 = jnp.zeros((_ZC, 128), jnp.float32)

    deg2 = _sc_deg(rows32, z16, ones16)[:, :_N, :]
    M1, dis = _tc1(deg2, X, W1)
    hr = _sc_segsum(4, cols16, rows16, z128, tuple(M1[i] for i in range(4)))[:, :_N]
    t, st = _tc2a(hr, dis)
    g4 = jnp.broadcast_to(gamma.reshape(4, 1, 128), (4, 8, 128))
    b4 = jnp.broadcast_to(beta.reshape(4, 1, 128), (4, 8, 128))
    M2 = _tc2b(t, st, g4, b4, dis, W2.reshape(4, 128, 256))
    zr = _sc_segsum(2, cols16, rows16, z128, (M2[0], M2[1]))
    dis_pad = jnp.concatenate(
        [dis, jnp.zeros((_PO - _N, 1), jnp.float32)])
    z = _tc3(zr, dis_pad)

    pad_d = _EDP2 - 2 * E
    A = jnp.concatenate([pos_edge_index[0], neg_edge_index[0],
                         jnp.zeros((pad_d,), jnp.int32)])
    B = jnp.concatenate([pos_edge_index[1], neg_edge_index[1],
                         jnp.zeros((pad_d,), jnp.int32)])
    part = _sc_decode(z[0], z[1],
                      A.reshape(_NS, _DB2, 128), B.reshape(_NS, _DB2, 128))
    sig = _tc4(part[:_EDP2], part[_EDP2:])
    return sig.reshape(-1)[: 2 * E].reshape(2, E)


# speedup vs baseline: 1.5219x; 1.5219x over previous
import functools

import jax
import jax.numpy as jnp
from jax import lax
from jax.experimental import pallas as pl
from jax.experimental.pallas import tpu as pltpu
from jax.experimental.pallas import tpu_sc as plsc

EPS = 1e-5
_N = 10000
_NP = 10240
_ZC = 640
_OC = 632
_PO = 16 * _OC
_RB = 1000
_NRB = _N // _RB
_NC, _NS = 2, 16
_NW = _NC * _NS

_EA_DEG_B = 42
_EAP = _NW * _EA_DEG_B * 128
_SEG_B = 84
_EDP2 = 16 * 160 * 128


def _mesh():
    return plsc.VectorSubcoreMesh(core_axis_name="c", subcore_axis_name="s")


def _tc1_body(deg2_ref, x_ref, w1_ref, m1_ref, dis_ref):
    degb = deg2_ref[0] + deg2_ref[1]
    dis = lax.rsqrt(degb[:, 0:1])
    dis_ref[...] = dis
    xs = x_ref[...] * dis
    m1_ref[0] = jnp.dot(xs, w1_ref[...], preferred_element_type=jnp.float32)


def _tc1(deg2, X, W1):
    return pl.pallas_call(
        _tc1_body,
        grid=(_NRB, 4),
        in_specs=[
            pl.BlockSpec((2, _RB, 16), lambda i, j: (0, i, 0)),
            pl.BlockSpec((_RB, 256), lambda i, j: (i, 0)),
            pl.BlockSpec((256, 128), lambda i, j: (0, j)),
        ],
        out_specs=[
            pl.BlockSpec((1, _RB, 128), lambda i, j: (j, i, 0)),
            pl.BlockSpec((_RB, 1), lambda i, j: (i, 0)),
        ],
        out_shape=[
            jax.ShapeDtypeStruct((4, _N, 128), jnp.float32),
            jax.ShapeDtypeStruct((_N, 1), jnp.float32),
        ],
    )(deg2, X, W1)


def _tc2a_body(hr_ref, dis_ref, t_ref, st_ref, acc):
    i = pl.program_id(1)
    t = (hr_ref[0] * dis_ref[...]) ** 2
    t_ref[0] = t

    @pl.when(i == 0)
    def _():
        acc[...] = jnp.zeros_like(acc)

    acc[0:1] += jnp.sum(t, axis=0, keepdims=True)
    acc[1:2] += jnp.sum(t * t, axis=0, keepdims=True)

    @pl.when(i == _NRB - 1)
    def _():
        st_ref[0] = acc[...]


def _tc2a(hr, dis):
    return pl.pallas_call(
        _tc2a_body,
        grid=(4, _NRB),
        in_specs=[
            pl.BlockSpec((1, _RB, 128), lambda k, i: (k, i, 0)),
            pl.BlockSpec((_RB, 1), lambda k, i: (i, 0)),
        ],
        out_specs=[
            pl.BlockSpec((1, _RB, 128), lambda k, i: (k, i, 0)),
            pl.BlockSpec((1, 8, 128), lambda k, i: (k, 0, 0)),
        ],
        out_shape=[
            jax.ShapeDtypeStruct((4, _N, 128), jnp.float32),
            jax.ShapeDtypeStruct((4, 8, 128), jnp.float32),
        ],
        scratch_shapes=[pltpu.VMEM((8, 128), jnp.float32)],
    )(hr, dis)


def _tc2b_body(t_ref, st_ref, g_ref, b_ref, dis_ref, w2_ref, m2_ref):
    k = pl.program_id(2)
    sm = st_ref[0, 0:1, :] * (1.0 / _N)
    sq = st_ref[0, 1:2, :] * (1.0 / _N)
    inv = lax.rsqrt(sq - sm * sm + EPS)
    hb = ((t_ref[0] - sm) * inv * g_ref[0, 0:1, :] + b_ref[0, 0:1, :]) * dis_ref[...]
    part = jnp.dot(hb, w2_ref[0], preferred_element_type=jnp.float32)

    @pl.when(k == 0)
    def _():
        m2_ref[0] = part

    @pl.when(k > 0)
    def _():
        m2_ref[0] += part


def _tc2b(t, st, gamma4, beta4, dis, w2r):
    return pl.pallas_call(
        _tc2b_body,
        grid=(_NRB, 2, 4),
        in_specs=[
            pl.BlockSpec((1, _RB, 128), lambda i, jo, k: (k, i, 0)),
            pl.BlockSpec((1, 8, 128), lambda i, jo, k: (k, 0, 0)),
            pl.BlockSpec((1, 8, 128), lambda i, jo, k: (k, 0, 0)),
            pl.BlockSpec((1, 8, 128), lambda i, jo, k: (k, 0, 0)),
            pl.BlockSpec((_RB, 1), lambda i, jo, k: (i, 0)),
            pl.BlockSpec((1, 128, 128), lambda i, jo, k: (k, 0, jo)),
        ],
        out_specs=pl.BlockSpec((1, _RB, 128), lambda i, jo, k: (jo, i, 0)),
        out_shape=jax.ShapeDtypeStruct((2, _N, 128), jnp.float32),
    )(t, st, gamma4, beta4, dis, w2r)


def _sc_deg(rows3, zeros16, ones16):
    @functools.partial(
        pl.kernel,
        out_type=jax.ShapeDtypeStruct((_NC * _PO, 16), jnp.float32),
        mesh=_mesh(),
        scratch_types=[
            pltpu.VMEM((_EA_DEG_B, 128), jnp.int32),
            pltpu.VMEM((128, 16), jnp.float32),
            pltpu.VMEM_SHARED((_NP, 16), jnp.float32),
        ],
    )
    def run(rows_h, z16_h, o16_h, deg_h, idx_v, ones_v, acc):
        c = lax.axis_index("c")
        s = lax.axis_index("s")
        wid = s * _NC + c
        pltpu.sync_copy(z16_h, acc.at[pl.ds(s * _ZC, _ZC)])
        pltpu.sync_copy(rows_h.at[wid], idx_v)
        pltpu.sync_copy(o16_h, ones_v)
        plsc.subcore_barrier()

        def bstep(b, carry):
            pltpu.sync_copy(ones_v, acc.at[idx_v.at[b]], add=True)
            return carry

        lax.fori_loop(0, _EA_DEG_B, bstep, 0)
        plsc.subcore_barrier()
        pltpu.sync_copy(acc.at[pl.ds(s * _OC, _OC)],
                        deg_h.at[pl.ds(c * _PO + s * _OC, _OC)])

    return run(rows3, zeros16, ones16).reshape(_NC, _PO, 16)


_SCH = 12


def _sc_segsum(nch, cols3, rows3, zerosb, mts):
    cpc = nch // _NC

    @functools.partial(
        pl.kernel,
        out_type=jax.ShapeDtypeStruct((nch * _PO, 128), jnp.float32),
        mesh=_mesh(),
        scratch_types=[
            pltpu.VMEM((_SCH, 128), jnp.int32),
            pltpu.VMEM((_SCH, 128), jnp.int32),
            pltpu.VMEM((128, 128), jnp.float32),
            pltpu.VMEM((128, 128), jnp.float32),
            pltpu.VMEM_SHARED((_NP, 128), jnp.float32),
            pltpu.SemaphoreType.DMA,
            pltpu.SemaphoreType.DMA,
            pltpu.SemaphoreType.DMA,
            pltpu.SemaphoreType.DMA,
        ],
    )
    def run(cols_h, rows_h, z_h, *rest):
        mt_hs = rest[:nch]
        out_h = rest[nch]
        cidx, ridx, g0, g1, acc, smg0, smg1, sms0, sms1 = rest[nch + 1:]
        c = lax.axis_index("c")
        s = lax.axis_index("s")
        for chunk in range(nch):
            my = (chunk // cpc) == c

            @pl.when(my)
            def _zero():
                pltpu.sync_copy(z_h, acc.at[pl.ds(s * _ZC, _ZC)])

            plsc.subcore_barrier()

            @pl.when(my)
            def _work():
                def q_step(q, cq):
                    cia = pltpu.async_copy(cols_h.at[s, q], cidx, smg0)
                    ria = pltpu.async_copy(rows_h.at[s, q], ridx, smg1)
                    cia.wait()
                    ria.wait()

                    def pair(i, cp):
                        b0 = 2 * i
                        b1 = 2 * i + 1
                        g0c = pltpu.async_copy(
                            mt_hs[chunk].at[cidx.at[b0]], g0, smg0)
                        g0c.wait()
                        g1c = pltpu.async_copy(
                            mt_hs[chunk].at[cidx.at[b1]], g1, smg1)
                        s0c = pltpu.async_copy(
                            g0, acc.at[ridx.at[b0]], sms0, add=True)
                        g1c.wait()
                        s0c.wait()
                        s1c = pltpu.async_copy(
                            g1, acc.at[ridx.at[b1]], sms1, add=True)
                        s1c.wait()
                        return cp

                    lax.fori_loop(0, _SCH // 2, pair, 0)
                    return cq

                lax.fori_loop(0, _SEG_B // _SCH, q_step, 0)

            plsc.subcore_barrier()

            @pl.when(my)
            def _out():
                pltpu.sync_copy(acc.at[pl.ds(s * _OC, _OC)],
                                out_h.at[pl.ds(chunk * _PO + s * _OC, _OC)])

    return run(cols3, rows3, zerosb, *mts).reshape(nch, _PO, 128)


def _tc3_body(zr_ref, dis_ref, z_ref):
    z_ref[0] = zr_ref[0] * dis_ref[...]


def _tc3(zr, dis_pad):
    return pl.pallas_call(
        _tc3_body,
        grid=(2, 16),
        in_specs=[
            pl.BlockSpec((1, _OC, 128), lambda k, i: (k, i, 0)),
            pl.BlockSpec((_OC, 1), lambda k, i: (i, 0)),
        ],
        out_specs=pl.BlockSpec((1, _OC, 128), lambda k, i: (k, i, 0)),
        out_shape=jax.ShapeDtypeStruct((2, _PO, 128), jnp.float32),
    )(zr, dis_pad)


_DB2 = 160
_DCH = 8
_EPS2 = _DB2 * 128
_ZT = _N - 15 * _OC


def _sc_decode(z0, z1, a3, b3):
    @functools.partial(
        pl.kernel,
        out_type=jax.ShapeDtypeStruct((_NC * _EDP2, 16), jnp.float32),
        mesh=_mesh(),
        scratch_types=[
            pltpu.VMEM((_DCH, 128), jnp.int32),
            pltpu.VMEM((_DCH, 128), jnp.int32),
            pltpu.VMEM((128, 128), jnp.float32),
            pltpu.VMEM((128, 128), jnp.float32),
            pltpu.VMEM((64, 16), jnp.float32),
            pltpu.VMEM_SHARED((_N, 128), jnp.float32),
            pltpu.SemaphoreType.DMA,
            pltpu.SemaphoreType.DMA,
        ],
    )
    def run(z0_h, z1_h, a_h, b_h, out_h, av, bv, ga, gb, prow, zsp,
            sema, semb):
        c = lax.axis_index("c")
        s = lax.axis_index("s")

        @pl.when(jnp.logical_and(c == 0, s < 15))
        def _():
            pltpu.sync_copy(z0_h.at[pl.ds(s * _OC, _OC)],
                            zsp.at[pl.ds(s * _OC, _OC)])

        @pl.when(jnp.logical_and(c == 0, s == 15))
        def _():
            pltpu.sync_copy(z0_h.at[pl.ds(15 * _OC, _ZT)],
                            zsp.at[pl.ds(15 * _OC, _ZT)])

        @pl.when(jnp.logical_and(c == 1, s < 15))
        def _():
            pltpu.sync_copy(z1_h.at[pl.ds(s * _OC, _OC)],
                            zsp.at[pl.ds(s * _OC, _OC)])

        @pl.when(jnp.logical_and(c == 1, s == 15))
        def _():
            pltpu.sync_copy(z1_h.at[pl.ds(15 * _OC, _ZT)],
                            zsp.at[pl.ds(15 * _OC, _ZT)])

        plsc.subcore_barrier()

        def chunk_step(ch, carry):
            ca = pltpu.async_copy(a_h.at[s, pl.ds(ch * _DCH, _DCH)], av, sema)
            cb = pltpu.async_copy(b_h.at[s, pl.ds(ch * _DCH, _DCH)], bv, semb)
            ca.wait()
            cb.wait()

            def batch(b, carry2):
                ga_c = pltpu.async_copy(zsp.at[av.at[b]], ga, sema)
                gb_c = pltpu.async_copy(zsp.at[bv.at[b]], gb, semb)
                ga_c.wait()
                gb_c.wait()

                for half in range(2):
                    def estep(e, cc):
                        eh = 64 * half + e
                        acc = ga[eh, pl.ds(0, 16)] * gb[eh, pl.ds(0, 16)]
                        for k in range(1, 8):
                            acc = acc + ga[eh, pl.ds(16 * k, 16)] * gb[eh, pl.ds(16 * k, 16)]
                        prow[e] = acc
                        return cc

                    lax.fori_loop(0, 64, estep, 0)
                    pltpu.sync_copy(
                        prow,
                        out_h.at[pl.ds(
                            c * _EDP2 + s * _EPS2
                            + 128 * (ch * _DCH + b) + 64 * half, 64)])
                return carry2

            lax.fori_loop(0, _DCH, batch, 0)
            return carry

        lax.fori_loop(0, _DB2 // _DCH, chunk_step, 0)

    return run(z0, z1, a3, b3)


_T4B = 2048


def _tc4_body(p0_ref, p1_ref, o_ref):
    tot = jnp.sum(p0_ref[...] + p1_ref[...], axis=1, keepdims=True)
    o_ref[...] = jax.nn.sigmoid(tot)


def _tc4(p0, p1):
    t = p0.shape[0]
    return pl.pallas_call(
        _tc4_body,
        grid=(t // _T4B,),
        in_specs=[
            pl.BlockSpec((_T4B, 16), lambda i: (i, 0)),
            pl.BlockSpec((_T4B, 16), lambda i: (i, 0)),
        ],
        out_specs=pl.BlockSpec((_T4B, 1), lambda i: (i, 0)),
        out_shape=jax.ShapeDtypeStruct((t, 1), jnp.float32),
    )(p0, p1)


def kernel(X, W1, W2, gamma, beta, adj_edge_index, pos_edge_index, neg_edge_index):
    E = pos_edge_index.shape[1]
    rows = adj_edge_index[0]
    cols = adj_edge_index[1]
    pad_a = _EAP - rows.shape[0]
    rows_p = jnp.concatenate([rows, jnp.full((pad_a,), _N, jnp.int32)])
    cols_p = jnp.concatenate([cols, jnp.zeros((pad_a,), jnp.int32)])
    rows32 = rows_p.reshape(_NW, _EA_DEG_B, 128)
    rows16 = rows_p.reshape(_NS, _SEG_B // _SCH, _SCH, 128)
    cols16 = cols_p.reshape(_NS, _SEG_B // _SCH, _SCH, 128)
    z16 = jnp.zeros((_ZC, 16), jnp.float32)
    ones16 = jnp.ones((128, 16), jnp.float32)
    z128 = jnp.zeros((_ZC, 128), jnp.float32)

    deg2 = _sc_deg(rows32, z16, ones16)[:, :_N, :]
    M1, dis = _tc1(deg2, X, W1)
    hr = _sc_segsum(4, cols16, rows16, z128, tuple(M1[i] for i in range(4)))[:, :_N]
    t, st = _tc2a(hr, dis)
    g4 = jnp.broadcast_to(gamma.reshape(4, 1, 128), (4, 8, 128))
    b4 = jnp.broadcast_to(beta.reshape(4, 1, 128), (4, 8, 128))
    M2 = _tc2b(t, st, g4, b4, dis, W2.reshape(4, 128, 256))
    zr = _sc_segsum(2, cols16, rows16, z128, (M2[0], M2[1]))
    dis_pad = jnp.concatenate(
        [dis, jnp.zeros((_PO - _N, 1), jnp.float32)])
    z = _tc3(zr, dis_pad)

    pad_d = _EDP2 - 2 * E
    A = jnp.concatenate([pos_edge_index[0], neg_edge_index[0],
                         jnp.zeros((pad_d,), jnp.int32)])
    B = jnp.concatenate([pos_edge_index[1], neg_edge_index[1],
                         jnp.zeros((pad_d,), jnp.int32)])
    part = _sc_decode(z[0], z[1],
                      A.reshape(_NS, _DB2, 128), B.reshape(_NS, _DB2, 128))
    sig = _tc4(part[:_EDP2], part[_EDP2:])
    return sig.reshape(-1)[: 2 * E].reshape(2, E)
